# baseline (speedup 1.0000x reference)
import functools

import jax
import jax.numpy as jnp
from jax import lax
from jax.experimental import pallas as pl
from jax.experimental.pallas import tpu as pltpu
from jax.experimental.pallas import tpu_sc as plsc

N = 10000
E = 320000
D = 128
H = 256

NC = 2
NS = 16
LANES = 16
NW = NC * NS
EPW = E // NW
CB = 80
NCHUNK = EPW // CB


def _bn_body(x_ref, g_ref, b_ref, m_ref, v_ref, o_ref):
    inv = g_ref[...] * lax.rsqrt(v_ref[...] + 1e-5)
    o_ref[...] = (x_ref[...] - m_ref[...]) * inv + b_ref[...]


def _batchnorm(x, gamma, beta, mean, var):
    blk = 2000
    return pl.pallas_call(
        _bn_body,
        grid=(N // blk,),
        in_specs=[
            pl.BlockSpec((blk, D), lambda i: (i, 0)),
            pl.BlockSpec((1, D), lambda i: (0, 0)),
            pl.BlockSpec((1, D), lambda i: (0, 0)),
            pl.BlockSpec((1, D), lambda i: (0, 0)),
            pl.BlockSpec((1, D), lambda i: (0, 0)),
        ],
        out_specs=pl.BlockSpec((blk, D), lambda i: (i, 0)),
        out_shape=jax.ShapeDtypeStruct((N, D), jnp.float32),
    )(x, gamma.reshape(1, D), beta.reshape(1, D), mean.reshape(1, D),
      var.reshape(1, D))


def _proj_body(ea_ref, w_ref, b_ref, o_ref):
    o_ref[...] = jnp.dot(ea_ref[...], w_ref[...],
                         preferred_element_type=jnp.float32) + b_ref[...]


def _edge_project(edge_attr, lin_W, lin_b):
    blk = 4000
    return pl.pallas_call(
        _proj_body,
        grid=(E // blk,),
        in_specs=[
            pl.BlockSpec((blk, D), lambda i: (i, 0)),
            pl.BlockSpec((D, D), lambda i: (0, 0)),
            pl.BlockSpec((1, D), lambda i: (0, 0)),
        ],
        out_specs=pl.BlockSpec((blk, D), lambda i: (i, 0)),
        out_shape=jax.ShapeDtypeStruct((E, D), jnp.float32),
    )(edge_attr, lin_W, lin_b.reshape(1, D))


def _sc_aggregate_body(h_hbm, e_hbm, src_hbm, dst_hbm, zeros_hbm, out_hbm,
                       acc, src_v, dst_v, ebuf, hbuf, ssem, dsem, esem, gsem):
    cid = lax.axis_index("c")
    sid = lax.axis_index("s")
    wid = sid * NC + cid
    w_base = wid * EPW

    rpt = 624
    rslice = pl.ds(sid * rpt, rpt)
    tail = pl.ds(NS * rpt, N - NS * rpt)
    pltpu.sync_copy(zeros_hbm.at[rslice, :], acc.at[rslice, :])

    @pl.when(sid == NS - 1)
    def _():
        pltpu.sync_copy(zeros_hbm.at[tail, :], acc.at[tail, :])

    plsc.subcore_barrier()

    def idx_copy(c, slot, ref, hbm, sem):
        pltpu.async_copy(hbm.at[pl.ds(w_base + c * CB, CB)], ref.at[slot], sem)

    def idx_wait(ref, hbm, sem):
        pltpu.make_async_copy(hbm.at[pl.ds(0, CB)], ref.at[0], sem).wait()

    def data_copy(c, slot):
        pltpu.async_copy(e_hbm.at[pl.ds(w_base + c * CB, CB), :],
                         ebuf.at[slot], esem)
        pltpu.async_copy(h_hbm.at[src_v.at[slot]], hbuf.at[slot], gsem)

    def data_wait(slot):
        pltpu.make_async_copy(e_hbm.at[pl.ds(0, CB), :], ebuf.at[slot],
                              esem).wait()
        pltpu.make_async_copy(h_hbm.at[src_v.at[0]], hbuf.at[slot],
                              gsem).wait()

    idx_copy(0, 0, src_v, src_hbm, ssem)
    idx_copy(0, 0, dst_v, dst_hbm, dsem)
    idx_copy(1, 1, src_v, src_hbm, ssem)
    idx_copy(1, 1, dst_v, dst_hbm, dsem)
    idx_wait(src_v, src_hbm, ssem)
    idx_wait(src_v, src_hbm, ssem)
    idx_wait(dst_v, dst_hbm, dsem)
    idx_wait(dst_v, dst_hbm, dsem)
    data_copy(0, 0)

    def chunk_body(c, carry):
        slot = lax.rem(c, 2)

        @pl.when(jnp.logical_and(c >= 1, c + 1 < NCHUNK))
        def _():
            idx_wait(src_v, src_hbm, ssem)

        @pl.when(c + 1 < NCHUNK)
        def _():
            data_copy(c + 1, 1 - slot)

        data_wait(slot)

        @pl.when(c + 2 < NCHUNK)
        def _():
            idx_copy(c + 2, slot, src_v, src_hbm, ssem)

        def row_body(r, rc):
            for j in range(D // LANES):
                sl = pl.ds(j * LANES, LANES)
                ebuf[slot, r, sl] = jnp.maximum(
                    ebuf[slot, r, sl] + hbuf[slot, r, sl], 0.0)
            return rc

        lax.fori_loop(0, CB, row_body, 0)

        @pl.when(c >= 2)
        def _():
            idx_wait(dst_v, dst_hbm, dsem)

        pltpu.sync_copy(ebuf.at[slot], acc.at[dst_v.at[slot]], add=True)

        @pl.when(c + 2 < NCHUNK)
        def _():
            idx_copy(c + 2, slot, dst_v, dst_hbm, dsem)

        return carry

    lax.fori_loop(0, NCHUNK, chunk_body, 0)
    plsc.subcore_barrier()
    pltpu.sync_copy(acc.at[rslice, :], out_hbm.at[cid, rslice, :])

    @pl.when(sid == NS - 1)
    def _():
        pltpu.sync_copy(acc.at[tail, :], out_hbm.at[cid, tail, :])


@functools.cache
def _sc_aggregate_fn():
    return pl.kernel(
        _sc_aggregate_body,
        mesh=plsc.VectorSubcoreMesh(core_axis_name="c", subcore_axis_name="s"),
        out_type=jax.ShapeDtypeStruct((NC, N, D), jnp.float32),
        scratch_types=[
            pltpu.VMEM_SHARED((N, D), jnp.float32),
            pltpu.VMEM((2, CB), jnp.int32),
            pltpu.VMEM((2, CB), jnp.int32),
            pltpu.VMEM((2, CB, D), jnp.float32),
            pltpu.VMEM((2, CB, D), jnp.float32),
            pltpu.SemaphoreType.DMA,
            pltpu.SemaphoreType.DMA,
            pltpu.SemaphoreType.DMA,
            pltpu.SemaphoreType.DMA,
        ],
    )


def _gelu_exact(v):
    return 0.5 * v * (1.0 + lax.erf(v * 0.7071067811865476))


def _mlp_body(x_ref, h_ref, a_ref, eps_ref, w1_ref, b1_ref,
              w2_ref, b2_ref, o_ref):
    eps = eps_ref[0]
    z = (1.0 + eps) * h_ref[...] + a_ref[0] + a_ref[1]
    hid = jnp.dot(z, w1_ref[...], preferred_element_type=jnp.float32) + b1_ref[...]
    hid = _gelu_exact(hid)
    oc = jnp.dot(hid, w2_ref[...], preferred_element_type=jnp.float32) + b2_ref[...]
    o_ref[...] = x_ref[...] + _gelu_exact(oc)


def _mlp_residual(x, h, agg, eps, W1, b1, W2, b2):
    blk = 2000
    return pl.pallas_call(
        _mlp_body,
        grid=(N // blk,),
        in_specs=[
            pl.BlockSpec((blk, D), lambda i: (i, 0)),
            pl.BlockSpec((blk, D), lambda i: (i, 0)),
            pl.BlockSpec((2, blk, D), lambda i: (0, i, 0)),
            pl.BlockSpec(memory_space=pltpu.SMEM),
            pl.BlockSpec((D, H), lambda i: (0, 0)),
            pl.BlockSpec((1, H), lambda i: (0, 0)),
            pl.BlockSpec((H, D), lambda i: (0, 0)),
            pl.BlockSpec((1, D), lambda i: (0, 0)),
        ],
        out_specs=pl.BlockSpec((blk, D), lambda i: (i, 0)),
        out_shape=jax.ShapeDtypeStruct((N, D), jnp.float32),
    )(x, h, agg, eps.reshape(1), W1, b1.reshape(1, H), W2, b2.reshape(1, D))


def kernel(x, edge_index, edge_attr, bn_gamma, bn_beta, bn_mean, bn_var, eps,
           lin_W, lin_b, W1, b1, W2, b2):
    h = _batchnorm(x, bn_gamma, bn_beta, bn_mean, bn_var)
    e = _edge_project(edge_attr, lin_W, lin_b)
    src = edge_index[0]
    dst = edge_index[1]
    zeros = jnp.zeros((N, D), dtype=jnp.float32)
    agg = _sc_aggregate_fn()(h, e, src, dst, zeros)
    return _mlp_residual(x, h, agg, eps, W1, b1, W2, b2)


# speedup vs baseline: 1.0621x; 1.0621x over previous
import functools

import jax
import jax.numpy as jnp
from jax import lax
from jax.experimental import pallas as pl
from jax.experimental.pallas import tpu as pltpu
from jax.experimental.pallas import tpu_sc as plsc

N = 10000
E = 320000
D = 128
H = 256

NC = 2
NS = 16
LANES = 16
NW = NC * NS
EPW = E // NW
CB = 80
NCHUNK = EPW // CB


def _bn_body(x_ref, g_ref, b_ref, m_ref, v_ref, o_ref):
    inv = g_ref[...] * lax.rsqrt(v_ref[...] + 1e-5)
    o_ref[...] = (x_ref[...] - m_ref[...]) * inv + b_ref[...]


def _batchnorm(x, gamma, beta, mean, var):
    blk = 2000
    return pl.pallas_call(
        _bn_body,
        grid=(N // blk,),
        in_specs=[
            pl.BlockSpec((blk, D), lambda i: (i, 0)),
            pl.BlockSpec((1, D), lambda i: (0, 0)),
            pl.BlockSpec((1, D), lambda i: (0, 0)),
            pl.BlockSpec((1, D), lambda i: (0, 0)),
            pl.BlockSpec((1, D), lambda i: (0, 0)),
        ],
        out_specs=pl.BlockSpec((blk, D), lambda i: (i, 0)),
        out_shape=jax.ShapeDtypeStruct((N, D), jnp.float32),
    )(x, gamma.reshape(1, D), beta.reshape(1, D), mean.reshape(1, D),
      var.reshape(1, D))


def _proj_body(ea_ref, w_ref, b_ref, o_ref):
    o_ref[...] = jnp.dot(ea_ref[...], w_ref[...],
                         preferred_element_type=jnp.float32) + b_ref[...]


def _edge_project(edge_attr, lin_W, lin_b):
    blk = 4000
    return pl.pallas_call(
        _proj_body,
        grid=(E // blk,),
        in_specs=[
            pl.BlockSpec((blk, D), lambda i: (i, 0)),
            pl.BlockSpec((D, D), lambda i: (0, 0)),
            pl.BlockSpec((1, D), lambda i: (0, 0)),
        ],
        out_specs=pl.BlockSpec((blk, D), lambda i: (i, 0)),
        out_shape=jax.ShapeDtypeStruct((E, D), jnp.float32),
    )(edge_attr, lin_W, lin_b.reshape(1, D))


def _sc_aggregate_body(h_hbm, e_hbm, src_hbm, dst_hbm, zeros_hbm, out_hbm,
                       acc, src_v, dst_v, ebuf, hbuf, ssem, dsem, esem, gsem):
    cid = lax.axis_index("c")
    sid = lax.axis_index("s")
    wid = sid * NC + cid
    w_base = wid * EPW

    rpt = 624
    rslice = pl.ds(sid * rpt, rpt)
    tail = pl.ds(NS * rpt, N - NS * rpt)
    pltpu.sync_copy(zeros_hbm.at[rslice, :], acc.at[rslice, :])

    @pl.when(sid == NS - 1)
    def _():
        pltpu.sync_copy(zeros_hbm.at[tail, :], acc.at[tail, :])

    plsc.subcore_barrier()

    def idx_copy(c, slot, ref, hbm, sem):
        pltpu.async_copy(hbm.at[pl.ds(w_base + c * CB, CB)], ref.at[slot], sem)

    def idx_wait(ref, hbm, sem):
        pltpu.make_async_copy(hbm.at[pl.ds(0, CB)], ref.at[0], sem).wait()

    def data_copy(c, slot):
        pltpu.async_copy(e_hbm.at[pl.ds(w_base + c * CB, CB), :],
                         ebuf.at[slot], esem)
        pltpu.async_copy(h_hbm.at[src_v.at[slot]], hbuf.at[slot], gsem)

    def data_wait(slot):
        pltpu.make_async_copy(e_hbm.at[pl.ds(0, CB), :], ebuf.at[slot],
                              esem).wait()
        pltpu.make_async_copy(h_hbm.at[src_v.at[0]], hbuf.at[slot],
                              gsem).wait()

    idx_copy(0, 0, src_v, src_hbm, ssem)
    idx_copy(0, 0, dst_v, dst_hbm, dsem)
    idx_copy(1, 1, src_v, src_hbm, ssem)
    idx_copy(1, 1, dst_v, dst_hbm, dsem)
    idx_wait(src_v, src_hbm, ssem)
    idx_wait(src_v, src_hbm, ssem)
    idx_wait(dst_v, dst_hbm, dsem)
    idx_wait(dst_v, dst_hbm, dsem)
    data_copy(0, 0)

    def chunk_body(c, carry):
        slot = lax.rem(c, 2)

        @pl.when(jnp.logical_and(c >= 1, c + 1 < NCHUNK))
        def _():
            idx_wait(src_v, src_hbm, ssem)

        @pl.when(c + 1 < NCHUNK)
        def _():
            data_copy(c + 1, 1 - slot)

        data_wait(slot)

        @pl.when(c + 2 < NCHUNK)
        def _():
            idx_copy(c + 2, slot, src_v, src_hbm, ssem)

        eb = ebuf.at[slot]
        hb = hbuf.at[slot]
        UNROLL = 8

        def row_body(r, rc):
            r0 = r * UNROLL
            for k in range(UNROLL):
                for j in range(D // LANES):
                    sl = pl.ds(j * LANES, LANES)
                    eb[r0 + k, sl] = jnp.maximum(
                        eb[r0 + k, sl] + hb[r0 + k, sl], 0.0)
            return rc

        lax.fori_loop(0, CB // UNROLL, row_body, 0)

        @pl.when(c >= 2)
        def _():
            idx_wait(dst_v, dst_hbm, dsem)

        pltpu.sync_copy(ebuf.at[slot], acc.at[dst_v.at[slot]], add=True)

        @pl.when(c + 2 < NCHUNK)
        def _():
            idx_copy(c + 2, slot, dst_v, dst_hbm, dsem)

        return carry

    lax.fori_loop(0, NCHUNK, chunk_body, 0)
    plsc.subcore_barrier()
    pltpu.sync_copy(acc.at[rslice, :], out_hbm.at[cid, rslice, :])

    @pl.when(sid == NS - 1)
    def _():
        pltpu.sync_copy(acc.at[tail, :], out_hbm.at[cid, tail, :])


@functools.cache
def _sc_aggregate_fn():
    return pl.kernel(
        _sc_aggregate_body,
        mesh=plsc.VectorSubcoreMesh(core_axis_name="c", subcore_axis_name="s"),
        out_type=jax.ShapeDtypeStruct((NC, N, D), jnp.float32),
        scratch_types=[
            pltpu.VMEM_SHARED((N, D), jnp.float32),
            pltpu.VMEM((2, CB), jnp.int32),
            pltpu.VMEM((2, CB), jnp.int32),
            pltpu.VMEM((2, CB, D), jnp.float32),
            pltpu.VMEM((2, CB, D), jnp.float32),
            pltpu.SemaphoreType.DMA,
            pltpu.SemaphoreType.DMA,
            pltpu.SemaphoreType.DMA,
            pltpu.SemaphoreType.DMA,
        ],
    )


def _gelu_exact(v):
    return 0.5 * v * (1.0 + lax.erf(v * 0.7071067811865476))


def _mlp_body(x_ref, h_ref, a_ref, eps_ref, w1_ref, b1_ref,
              w2_ref, b2_ref, o_ref):
    eps = eps_ref[0]
    z = (1.0 + eps) * h_ref[...] + a_ref[0] + a_ref[1]
    hid = jnp.dot(z, w1_ref[...], preferred_element_type=jnp.float32) + b1_ref[...]
    hid = _gelu_exact(hid)
    oc = jnp.dot(hid, w2_ref[...], preferred_element_type=jnp.float32) + b2_ref[...]
    o_ref[...] = x_ref[...] + _gelu_exact(oc)


def _mlp_residual(x, h, agg, eps, W1, b1, W2, b2):
    blk = 2000
    return pl.pallas_call(
        _mlp_body,
        grid=(N // blk,),
        in_specs=[
            pl.BlockSpec((blk, D), lambda i: (i, 0)),
            pl.BlockSpec((blk, D), lambda i: (i, 0)),
            pl.BlockSpec((2, blk, D), lambda i: (0, i, 0)),
            pl.BlockSpec(memory_space=pltpu.SMEM),
            pl.BlockSpec((D, H), lambda i: (0, 0)),
            pl.BlockSpec((1, H), lambda i: (0, 0)),
            pl.BlockSpec((H, D), lambda i: (0, 0)),
            pl.BlockSpec((1, D), lambda i: (0, 0)),
        ],
        out_specs=pl.BlockSpec((blk, D), lambda i: (i, 0)),
        out_shape=jax.ShapeDtypeStruct((N, D), jnp.float32),
    )(x, h, agg, eps.reshape(1), W1, b1.reshape(1, H), W2, b2.reshape(1, D))


def kernel(x, edge_index, edge_attr, bn_gamma, bn_beta, bn_mean, bn_var, eps,
           lin_W, lin_b, W1, b1, W2, b2):
    h = _batchnorm(x, bn_gamma, bn_beta, bn_mean, bn_var)
    e = _edge_project(edge_attr, lin_W, lin_b)
    src = edge_index[0]
    dst = edge_index[1]
    zeros = jnp.zeros((N, D), dtype=jnp.float32)
    agg = _sc_aggregate_fn()(h, e, src, dst, zeros)
    return _mlp_residual(x, h, agg, eps, W1, b1, W2, b2)


# speedup vs baseline: 2.0391x; 1.9198x over previous
import functools

import jax
import jax.numpy as jnp
from jax import lax
from jax.experimental import pallas as pl
from jax.experimental.pallas import tpu as pltpu
from jax.experimental.pallas import tpu_sc as plsc

N = 10000
E = 320000
D = 128
H = 256

NC = 2
NS = 16
LANES = 16
NW = NC * NS
EPW = E // NW
CB = 80
NCHUNK = EPW // CB


def _bn_body(x_ref, g_ref, b_ref, m_ref, v_ref, o_ref):
    inv = g_ref[...] * lax.rsqrt(v_ref[...] + 1e-5)
    o_ref[...] = (x_ref[...] - m_ref[...]) * inv + b_ref[...]


def _batchnorm(x, gamma, beta, mean, var):
    blk = 2000
    return pl.pallas_call(
        _bn_body,
        grid=(N // blk,),
        in_specs=[
            pl.BlockSpec((blk, D), lambda i: (i, 0)),
            pl.BlockSpec((1, D), lambda i: (0, 0)),
            pl.BlockSpec((1, D), lambda i: (0, 0)),
            pl.BlockSpec((1, D), lambda i: (0, 0)),
            pl.BlockSpec((1, D), lambda i: (0, 0)),
        ],
        out_specs=pl.BlockSpec((blk, D), lambda i: (i, 0)),
        out_shape=jax.ShapeDtypeStruct((N, D), jnp.float32),
    )(x, gamma.reshape(1, D), beta.reshape(1, D), mean.reshape(1, D),
      var.reshape(1, D))


def _proj_body(ea_ref, w_ref, b_ref, o_ref):
    o_ref[...] = jnp.dot(ea_ref[...], w_ref[...],
                         preferred_element_type=jnp.float32) + b_ref[...]


def _edge_project(edge_attr, lin_W, lin_b):
    blk = 4000
    return pl.pallas_call(
        _proj_body,
        grid=(E // blk,),
        in_specs=[
            pl.BlockSpec((blk, D), lambda i: (i, 0)),
            pl.BlockSpec((D, D), lambda i: (0, 0)),
            pl.BlockSpec((1, D), lambda i: (0, 0)),
        ],
        out_specs=pl.BlockSpec((blk, D), lambda i: (i, 0)),
        out_shape=jax.ShapeDtypeStruct((E, D), jnp.float32),
    )(edge_attr, lin_W, lin_b.reshape(1, D))


def _sc_aggregate_body(h_hbm, e_hbm, src_hbm, dst_hbm, zeros_hbm, out_hbm,
                       acc, src_v, dst_v, ebuf, hbuf, ssem, dsem, esem, gsem):
    cid = lax.axis_index("c")
    sid = lax.axis_index("s")
    wid = sid * NC + cid
    w_base = wid * EPW

    rpt = 624
    rslice = pl.ds(sid * rpt, rpt)
    tail = pl.ds(NS * rpt, N - NS * rpt)
    pltpu.sync_copy(zeros_hbm.at[rslice, :], acc.at[rslice, :])

    @pl.when(sid == NS - 1)
    def _():
        pltpu.sync_copy(zeros_hbm.at[tail, :], acc.at[tail, :])

    plsc.subcore_barrier()

    def idx_copy(c, slot, ref, hbm, sem):
        pltpu.async_copy(hbm.at[pl.ds(w_base + c * CB, CB)], ref.at[slot], sem)

    def idx_wait(ref, hbm, sem):
        pltpu.make_async_copy(hbm.at[pl.ds(0, CB)], ref.at[0], sem).wait()

    def data_copy(c, slot):
        pltpu.async_copy(e_hbm.at[pl.ds(w_base + c * CB, CB), :],
                         ebuf.at[slot], esem)
        pltpu.async_copy(h_hbm.at[src_v.at[slot]], hbuf.at[slot], gsem)

    def data_wait(slot):
        pltpu.make_async_copy(e_hbm.at[pl.ds(0, CB), :], ebuf.at[slot],
                              esem).wait()
        pltpu.make_async_copy(h_hbm.at[src_v.at[0]], hbuf.at[slot],
                              gsem).wait()

    idx_copy(0, 0, src_v, src_hbm, ssem)
    idx_copy(0, 0, dst_v, dst_hbm, dsem)
    idx_copy(1, 1, src_v, src_hbm, ssem)
    idx_copy(1, 1, dst_v, dst_hbm, dsem)
    idx_wait(src_v, src_hbm, ssem)
    idx_wait(src_v, src_hbm, ssem)
    idx_wait(dst_v, dst_hbm, dsem)
    idx_wait(dst_v, dst_hbm, dsem)
    data_copy(0, 0)

    def chunk_body(c, carry):
        slot = lax.rem(c, 2)

        @pl.when(jnp.logical_and(c >= 1, c + 1 < NCHUNK))
        def _():
            idx_wait(src_v, src_hbm, ssem)

        @pl.when(c + 1 < NCHUNK)
        def _():
            data_copy(c + 1, 1 - slot)

        data_wait(slot)

        @pl.when(c + 2 < NCHUNK)
        def _():
            idx_copy(c + 2, slot, src_v, src_hbm, ssem)

        eb = ebuf.at[slot]
        hb = hbuf.at[slot]

        @plsc.parallel_loop(0, CB, step=1, unroll=4)
        def _(r):
            for j in range(D // LANES):
                sl = pl.ds(j * LANES, LANES)
                eb[r, sl] = jnp.maximum(eb[r, sl] + hb[r, sl], 0.0)

        @pl.when(c >= 2)
        def _():
            idx_wait(dst_v, dst_hbm, dsem)

        pltpu.sync_copy(ebuf.at[slot], acc.at[dst_v.at[slot]], add=True)

        @pl.when(c + 2 < NCHUNK)
        def _():
            idx_copy(c + 2, slot, dst_v, dst_hbm, dsem)

        return carry

    lax.fori_loop(0, NCHUNK, chunk_body, 0)
    plsc.subcore_barrier()
    pltpu.sync_copy(acc.at[rslice, :], out_hbm.at[cid, rslice, :])

    @pl.when(sid == NS - 1)
    def _():
        pltpu.sync_copy(acc.at[tail, :], out_hbm.at[cid, tail, :])


@functools.cache
def _sc_aggregate_fn():
    return pl.kernel(
        _sc_aggregate_body,
        mesh=plsc.VectorSubcoreMesh(core_axis_name="c", subcore_axis_name="s"),
        out_type=jax.ShapeDtypeStruct((NC, N, D), jnp.float32),
        scratch_types=[
            pltpu.VMEM_SHARED((N, D), jnp.float32),
            pltpu.VMEM((2, CB), jnp.int32),
            pltpu.VMEM((2, CB), jnp.int32),
            pltpu.VMEM((2, CB, D), jnp.float32),
            pltpu.VMEM((2, CB, D), jnp.float32),
            pltpu.SemaphoreType.DMA,
            pltpu.SemaphoreType.DMA,
            pltpu.SemaphoreType.DMA,
            pltpu.SemaphoreType.DMA,
        ],
    )


def _gelu_exact(v):
    return 0.5 * v * (1.0 + lax.erf(v * 0.7071067811865476))


def _mlp_body(x_ref, h_ref, a_ref, eps_ref, w1_ref, b1_ref,
              w2_ref, b2_ref, o_ref):
    eps = eps_ref[0]
    z = (1.0 + eps) * h_ref[...] + a_ref[0] + a_ref[1]
    hid = jnp.dot(z, w1_ref[...], preferred_element_type=jnp.float32) + b1_ref[...]
    hid = _gelu_exact(hid)
    oc = jnp.dot(hid, w2_ref[...], preferred_element_type=jnp.float32) + b2_ref[...]
    o_ref[...] = x_ref[...] + _gelu_exact(oc)


def _mlp_residual(x, h, agg, eps, W1, b1, W2, b2):
    blk = 2000
    return pl.pallas_call(
        _mlp_body,
        grid=(N // blk,),
        in_specs=[
            pl.BlockSpec((blk, D), lambda i: (i, 0)),
            pl.BlockSpec((blk, D), lambda i: (i, 0)),
            pl.BlockSpec((2, blk, D), lambda i: (0, i, 0)),
            pl.BlockSpec(memory_space=pltpu.SMEM),
            pl.BlockSpec((D, H), lambda i: (0, 0)),
            pl.BlockSpec((1, H), lambda i: (0, 0)),
            pl.BlockSpec((H, D), lambda i: (0, 0)),
            pl.BlockSpec((1, D), lambda i: (0, 0)),
        ],
        out_specs=pl.BlockSpec((blk, D), lambda i: (i, 0)),
        out_shape=jax.ShapeDtypeStruct((N, D), jnp.float32),
    )(x, h, agg, eps.reshape(1), W1, b1.reshape(1, H), W2, b2.reshape(1, D))


def kernel(x, edge_index, edge_attr, bn_gamma, bn_beta, bn_mean, bn_var, eps,
           lin_W, lin_b, W1, b1, W2, b2):
    h = _batchnorm(x, bn_gamma, bn_beta, bn_mean, bn_var)
    e = _edge_project(edge_attr, lin_W, lin_b)
    src = edge_index[0]
    dst = edge_index[1]
    zeros = jnp.zeros((N, D), dtype=jnp.float32)
    agg = _sc_aggregate_fn()(h, e, src, dst, zeros)
    return _mlp_residual(x, h, agg, eps, W1, b1, W2, b2)
